# baseline (speedup 1.0000x reference)
import functools
import jax
import jax.numpy as jnp
from jax import lax
from jax.experimental import pallas as pl
from jax.experimental.pallas import tpu as pltpu
from jax.experimental.pallas import tpu_sc as plsc

N_NODES = 10000
N_EDGES = 320000
D_NODE = 128
D_EDGE = 16

NC = 2
NS = 16
NW = NC * NS
EPAD = 327680
NCHUNK = 4
HALF = EPAD // NCHUNK
EPWH = HALF // NW
NPAD = 10112
KB = 10
BLK = KB * 128
RPT = NPAD // NS
EBLK = 8192


def _node_proj_body(x_ref, w_ref, ps_ref, pd_ref):
    p = jnp.dot(x_ref[...], w_ref[...], preferred_element_type=jnp.float32)
    ps_ref[...] = p[:, :D_EDGE]
    pd_ref[...] = p[:, D_EDGE:]


def _node_proj(x_pad, w_sd):
    return pl.pallas_call(
        _node_proj_body,
        out_shape=(
            jax.ShapeDtypeStruct((NPAD, D_EDGE), jnp.float32),
            jax.ShapeDtypeStruct((NPAD, D_EDGE), jnp.float32),
        ),
    )(x_pad, w_sd)


def _edge_proj_packed_body(a_ref, w_ref, b_ref, o_ref):
    o_ref[...] = (
        jnp.dot(a_ref[...], w_ref[...], preferred_element_type=jnp.float32)
        + b_ref[...]
    )


def _edge_proj_packed(a_p, w_blk, b_tiled):
    rows = a_p.shape[0]
    return pl.pallas_call(
        _edge_proj_packed_body,
        grid=(rows // (EBLK // 8),),
        in_specs=[
            pl.BlockSpec((EBLK // 8, 128), lambda i: (i, 0)),
            pl.BlockSpec((128, 128), lambda i: (0, 0)),
            pl.BlockSpec((1, 128), lambda i: (0, 0)),
        ],
        out_specs=pl.BlockSpec((EBLK // 8, 128), lambda i: (i, 0)),
        out_shape=jax.ShapeDtypeStruct((rows, 128), jnp.float32),
    )(a_p, w_blk, b_tiled.reshape(1, 128))


def _layer1_prep_body(*refs):
    agg_refs = refs[:NCHUNK]
    x_ref, wnx_ref, wna_ref, bn_ref, wsd_ref, ps_ref, pd_ref = refs[NCHUNK:]
    agg = sum(r[0] + r[1] for r in agg_refs)
    x1 = jnp.maximum(
        jnp.dot(x_ref[...], wnx_ref[...], preferred_element_type=jnp.float32)
        + jnp.dot(agg, wna_ref[...], preferred_element_type=jnp.float32)
        + bn_ref[...],
        0.0,
    )
    p = jnp.dot(x1, wsd_ref[...], preferred_element_type=jnp.float32)
    ps_ref[...] = p[:, :D_EDGE]
    pd_ref[...] = p[:, D_EDGE:]


def _layer1_prep(agg2s, x_pad, wnx, wna, bn, w_sd):
    return pl.pallas_call(
        _layer1_prep_body,
        out_shape=(
            jax.ShapeDtypeStruct((NPAD, D_EDGE), jnp.float32),
            jax.ShapeDtypeStruct((NPAD, D_EDGE), jnp.float32),
        ),
    )(*agg2s, x_pad, wnx, wna, bn.reshape(1, D_NODE), w_sd)


def _mlp_body(e_ref, m0_ref, b0_ref, m1_ref, b1_ref, m2_ref, b2_ref, o_ref):
    h = jnp.maximum(
        jnp.dot(e_ref[...], m0_ref[...], preferred_element_type=jnp.float32)
        + b0_ref[...], 0.0)
    h = jnp.maximum(
        jnp.dot(h, m1_ref[...], preferred_element_type=jnp.float32)
        + b1_ref[...], 0.0)
    o_ref[...] = (
        jnp.dot(h, m2_ref[...], preferred_element_type=jnp.float32)
        + b2_ref[...]
    )


def _mlp_head(e2_p, m0_blk, b0_t, m1_blk, b1_t, m2_blk, b2_t):
    rows = e2_p.shape[0]
    return pl.pallas_call(
        _mlp_body,
        grid=(rows // (EBLK // 8),),
        in_specs=[
            pl.BlockSpec((EBLK // 8, 128), lambda i: (i, 0)),
            pl.BlockSpec((128, 128), lambda i: (0, 0)),
            pl.BlockSpec((1, 128), lambda i: (0, 0)),
            pl.BlockSpec((128, 128), lambda i: (0, 0)),
            pl.BlockSpec((1, 128), lambda i: (0, 0)),
            pl.BlockSpec((128, 8), lambda i: (0, 0)),
            pl.BlockSpec((1, 8), lambda i: (0, 0)),
        ],
        out_specs=pl.BlockSpec((EBLK // 8, 8), lambda i: (i, 0)),
        out_shape=jax.ShapeDtypeStruct((rows, 8), jnp.float32),
    )(e2_p, m0_blk, b0_t.reshape(1, 128), m1_blk, b1_t.reshape(1, 128),
      m2_blk, b2_t.reshape(1, 8))


_MESH = plsc.VectorSubcoreMesh(core_axis_name="c", subcore_axis_name="s")


def _edge_sweep(wid, src_hbm, dst_hbm, ps_sh, pd_sh, ea_hbm, out_hbm,
                idx_s, idx_d, rows_s, rows_d, acc, sem, agg_sh=None):
    idx_row0 = wid * (EPWH // 128)

    def _step(t, carry):
        r0 = idx_row0 + t * KB
        pltpu.sync_copy(src_hbm.at[pl.ds(r0, KB)], idx_s)
        pltpu.sync_copy(dst_hbm.at[pl.ds(r0, KB)], idx_d)
        cps = []
        for j in range(KB):
            cps.append(pltpu.async_copy(
                ps_sh.at[idx_s.at[j]], rows_s.at[pl.ds(j * 128, 128)], sem))
            cps.append(pltpu.async_copy(
                pd_sh.at[idx_d.at[j]], rows_d.at[pl.ds(j * 128, 128)], sem))
        ebase = wid * EPWH + t * BLK
        pltpu.sync_copy(ea_hbm.at[pl.ds(ebase, BLK)], acc)
        for c in cps:
            c.wait()

        def _compute(i, carry2):
            acc[i] = jnp.maximum(acc[i] + rows_s[i] + rows_d[i], 0.0)
            return carry2
        lax.fori_loop(0, BLK, _compute, 0, unroll=4)

        pltpu.sync_copy(acc, out_hbm.at[pl.ds(ebase, BLK)])
        if agg_sh is not None:
            for j in range(KB):
                pltpu.sync_copy(acc.at[pl.ds(j * 128, 128)],
                                agg_sh.at[idx_d.at[j]], add=True)
        return carry
    lax.fori_loop(0, EPWH // BLK, _step, 0)


def _sc_phase1_body(src_hbm, dst_hbm, ps_hbm, pd_hbm, ea_hbm,
                    e1_hbm, agg2_hbm,
                    idx_s, idx_d, rows_s, rows_d, acc,
                    ps_sh, pd_sh, agg_sh, sem):
    cid = lax.axis_index("c")
    sid = lax.axis_index("s")
    wid = sid * NC + cid

    pltpu.sync_copy(ps_hbm.at[pl.ds(sid * RPT, RPT)],
                    ps_sh.at[pl.ds(sid * RPT, RPT)])
    pltpu.sync_copy(pd_hbm.at[pl.ds(sid * RPT, RPT)],
                    pd_sh.at[pl.ds(sid * RPT, RPT)])
    def _zero(i, carry):
        acc[i] = jnp.zeros((16,), jnp.float32)
        return carry
    lax.fori_loop(0, RPT, _zero, 0, unroll=4)
    pltpu.sync_copy(acc.at[pl.ds(0, RPT)], agg_sh.at[pl.ds(sid * RPT, RPT)])
    plsc.subcore_barrier()

    _edge_sweep(wid, src_hbm, dst_hbm, ps_sh, pd_sh, ea_hbm, e1_hbm,
                idx_s, idx_d, rows_s, rows_d, acc, sem, agg_sh=agg_sh)

    plsc.subcore_barrier()
    pltpu.sync_copy(agg_sh.at[pl.ds(sid * RPT, RPT)], acc.at[pl.ds(0, RPT)])
    pltpu.sync_copy(acc.at[pl.ds(0, RPT)],
                    agg2_hbm.at[cid, pl.ds(sid * RPT, RPT)])


_sc_phase1 = pl.kernel(
    _sc_phase1_body,
    out_type=(
        jax.ShapeDtypeStruct((HALF, D_EDGE), jnp.float32),
        jax.ShapeDtypeStruct((NC, NPAD, D_EDGE), jnp.float32),
    ),
    mesh=_MESH,
    scratch_types=[
        pltpu.VMEM((KB, 128), jnp.int32),
        pltpu.VMEM((KB, 128), jnp.int32),
        pltpu.VMEM((BLK, D_EDGE), jnp.float32),
        pltpu.VMEM((BLK, D_EDGE), jnp.float32),
        pltpu.VMEM((BLK, D_EDGE), jnp.float32),
        pltpu.VMEM_SHARED((NPAD, D_EDGE), jnp.float32),
        pltpu.VMEM_SHARED((NPAD, D_EDGE), jnp.float32),
        pltpu.VMEM_SHARED((NPAD, D_EDGE), jnp.float32),
        pltpu.SemaphoreType.DMA,
    ],
    compiler_params=pltpu.CompilerParams(use_tc_tiling_on_sc=False),
)


def _sc_phase2_body(src_hbm, dst_hbm, ps_hbm, pd_hbm, ee_hbm, e2_hbm,
                    idx_s, idx_d, rows_s, rows_d, acc, ps_sh, pd_sh, sem):
    cid = lax.axis_index("c")
    sid = lax.axis_index("s")
    wid = sid * NC + cid
    pltpu.sync_copy(ps_hbm.at[pl.ds(sid * RPT, RPT)],
                    ps_sh.at[pl.ds(sid * RPT, RPT)])
    pltpu.sync_copy(pd_hbm.at[pl.ds(sid * RPT, RPT)],
                    pd_sh.at[pl.ds(sid * RPT, RPT)])
    plsc.subcore_barrier()

    _edge_sweep(wid, src_hbm, dst_hbm, ps_sh, pd_sh, ee_hbm, e2_hbm,
                idx_s, idx_d, rows_s, rows_d, acc, sem)


_sc_phase2 = pl.kernel(
    _sc_phase2_body,
    out_type=jax.ShapeDtypeStruct((HALF, D_EDGE), jnp.float32),
    mesh=_MESH,
    scratch_types=[
        pltpu.VMEM((KB, 128), jnp.int32),
        pltpu.VMEM((KB, 128), jnp.int32),
        pltpu.VMEM((BLK, D_EDGE), jnp.float32),
        pltpu.VMEM((BLK, D_EDGE), jnp.float32),
        pltpu.VMEM((BLK, D_EDGE), jnp.float32),
        pltpu.VMEM_SHARED((NPAD, D_EDGE), jnp.float32),
        pltpu.VMEM_SHARED((NPAD, D_EDGE), jnp.float32),
        pltpu.SemaphoreType.DMA,
    ],
    compiler_params=pltpu.CompilerParams(use_tc_tiling_on_sc=False),
)


@jax.jit
def kernel(edge_index, x, edge_attr,
           We0, be0, Wn0, bn0, We1, be1, Wn1, bn1,
           M0, bm0, M1, bm1, M2, bm2):
    x = x.astype(jnp.float32)
    pad_e = EPAD - N_EDGES
    src = jnp.concatenate(
        [edge_index[0], jnp.full((pad_e,), N_NODES, jnp.int32)]
    ).reshape(EPAD // 128, 128)
    dst = jnp.concatenate(
        [edge_index[1], jnp.full((pad_e,), N_NODES, jnp.int32)]
    ).reshape(EPAD // 128, 128)
    hr = HALF // 128
    srch = tuple(src[k * hr:(k + 1) * hr] for k in range(NCHUNK))
    dsth = tuple(dst[k * hr:(k + 1) * hr] for k in range(NCHUNK))
    x_pad = jnp.pad(x, ((0, NPAD - N_NODES), (0, 0)))
    ea_ph = tuple(
        edge_attr[k * HALF:(k + 1) * HALF].reshape(HALF // 8, 128)
        if (k + 1) * HALF <= N_EDGES else
        jnp.pad(edge_attr[k * HALF:].reshape((N_EDGES - k * HALF) // 8, 128),
                ((0, ((k + 1) * HALF - N_EDGES) // 8), (0, 0)))
        for k in range(NCHUNK)
    )

    w0sd = jnp.concatenate([We0[:D_NODE], We0[D_NODE:2 * D_NODE]], axis=1)
    w1sd = jnp.concatenate([We1[:D_NODE], We1[D_NODE:2 * D_NODE]], axis=1)
    eye8 = jnp.eye(8, dtype=jnp.float32)
    w0e = jnp.kron(eye8, We0[2 * D_NODE:])
    b0e = jnp.tile(be0, 8)
    w1e = jnp.kron(eye8, We1[2 * D_NODE:])
    b1e = jnp.tile(be1, 8)

    km0 = jnp.kron(eye8, M0)
    km1 = jnp.kron(eye8, M1)
    km2 = jnp.kron(eye8, M2)
    bt0 = jnp.tile(bm0, 8)
    bt1 = jnp.tile(bm1, 8)
    bt2 = jnp.tile(bm2, 8)

    p0s, p0d = _node_proj(x_pad, w0sd)
    ea0_h = [_edge_proj_packed(ea_ph[k], w0e, b0e) for k in range(NCHUNK)]
    e1_h = [None] * NCHUNK
    agg2_h = [None] * NCHUNK
    for k in range(NCHUNK):
        e1_h[k], agg2_h[k] = _sc_phase1(
            srch[k], dsth[k], p0s, p0d, ea0_h[k].reshape(HALF, D_EDGE))
    p1s, p1d = _layer1_prep(agg2_h, x_pad,
                            Wn0[:D_NODE], Wn0[D_NODE:], bn0, w1sd)
    ee1_h = [_edge_proj_packed(e1_h[k].reshape(HALF // 8, 128), w1e, b1e)
             for k in range(NCHUNK)]
    out_h = []
    for k in range(NCHUNK):
        e2 = _sc_phase2(srch[k], dsth[k], p1s, p1d,
                        ee1_h[k].reshape(HALF, D_EDGE))
        out_h.append(_mlp_head(e2.reshape(HALF // 8, 128),
                               km0, bt0, km1, bt1, km2, bt2))
    out = jnp.concatenate(out_h, axis=0)
    return out.reshape(EPAD)[:N_EDGES]


# speedup vs baseline: 1.1484x; 1.1484x over previous
import functools
import jax
import jax.numpy as jnp
from jax import lax
from jax.experimental import pallas as pl
from jax.experimental.pallas import tpu as pltpu
from jax.experimental.pallas import tpu_sc as plsc

N_NODES = 10000
N_EDGES = 320000
D_NODE = 128
D_EDGE = 16

NC = 2
NS = 16
NW = NC * NS
EPAD = 327680
NCHUNK = 2
HALF = EPAD // NCHUNK
EPWH = HALF // NW
NPAD = 10112
KB = 10
BLK = KB * 128
RPT = NPAD // NS
EBLK = 8192


def _node_proj_body(x_ref, w_ref, ps_ref, pd_ref):
    p = jnp.dot(x_ref[...], w_ref[...], preferred_element_type=jnp.float32)
    ps_ref[...] = p[:, :D_EDGE]
    pd_ref[...] = p[:, D_EDGE:]


def _node_proj(x_pad, w_sd):
    return pl.pallas_call(
        _node_proj_body,
        out_shape=(
            jax.ShapeDtypeStruct((NPAD, D_EDGE), jnp.float32),
            jax.ShapeDtypeStruct((NPAD, D_EDGE), jnp.float32),
        ),
    )(x_pad, w_sd)


def _edge_proj_packed_body(a_ref, w_ref, b_ref, o_ref):
    o_ref[...] = (
        jnp.dot(a_ref[...], w_ref[...], preferred_element_type=jnp.float32)
        + b_ref[...]
    )


def _edge_proj_packed(a_p, w_blk, b_tiled):
    rows = a_p.shape[0]
    return pl.pallas_call(
        _edge_proj_packed_body,
        grid=(rows // (EBLK // 8),),
        in_specs=[
            pl.BlockSpec((EBLK // 8, 128), lambda i: (i, 0)),
            pl.BlockSpec((128, 128), lambda i: (0, 0)),
            pl.BlockSpec((1, 128), lambda i: (0, 0)),
        ],
        out_specs=pl.BlockSpec((EBLK // 8, 128), lambda i: (i, 0)),
        out_shape=jax.ShapeDtypeStruct((rows, 128), jnp.float32),
    )(a_p, w_blk, b_tiled.reshape(1, 128))


def _layer1_prep_body(*refs):
    agg_refs = refs[:NCHUNK]
    x_ref, wnx_ref, wna_ref, bn_ref, wsd_ref, ps_ref, pd_ref = refs[NCHUNK:]
    agg = sum(r[0] + r[1] for r in agg_refs)
    x1 = jnp.maximum(
        jnp.dot(x_ref[...], wnx_ref[...], preferred_element_type=jnp.float32)
        + jnp.dot(agg, wna_ref[...], preferred_element_type=jnp.float32)
        + bn_ref[...],
        0.0,
    )
    p = jnp.dot(x1, wsd_ref[...], preferred_element_type=jnp.float32)
    ps_ref[...] = p[:, :D_EDGE]
    pd_ref[...] = p[:, D_EDGE:]


def _layer1_prep(agg2s, x_pad, wnx, wna, bn, w_sd):
    return pl.pallas_call(
        _layer1_prep_body,
        out_shape=(
            jax.ShapeDtypeStruct((NPAD, D_EDGE), jnp.float32),
            jax.ShapeDtypeStruct((NPAD, D_EDGE), jnp.float32),
        ),
    )(*agg2s, x_pad, wnx, wna, bn.reshape(1, D_NODE), w_sd)


def _mlp_body(e_ref, m0_ref, b0_ref, m1_ref, b1_ref, m2_ref, b2_ref, o_ref):
    h = jnp.maximum(
        jnp.dot(e_ref[...], m0_ref[...], preferred_element_type=jnp.float32)
        + b0_ref[...], 0.0)
    h = jnp.maximum(
        jnp.dot(h, m1_ref[...], preferred_element_type=jnp.float32)
        + b1_ref[...], 0.0)
    o_ref[...] = (
        jnp.dot(h, m2_ref[...], preferred_element_type=jnp.float32)
        + b2_ref[...]
    )


def _mlp_head(e2_p, m0_blk, b0_t, m1_blk, b1_t, m2_blk, b2_t):
    rows = e2_p.shape[0]
    return pl.pallas_call(
        _mlp_body,
        grid=(rows // (EBLK // 8),),
        in_specs=[
            pl.BlockSpec((EBLK // 8, 128), lambda i: (i, 0)),
            pl.BlockSpec((128, 128), lambda i: (0, 0)),
            pl.BlockSpec((1, 128), lambda i: (0, 0)),
            pl.BlockSpec((128, 128), lambda i: (0, 0)),
            pl.BlockSpec((1, 128), lambda i: (0, 0)),
            pl.BlockSpec((128, 8), lambda i: (0, 0)),
            pl.BlockSpec((1, 8), lambda i: (0, 0)),
        ],
        out_specs=pl.BlockSpec((EBLK // 8, 8), lambda i: (i, 0)),
        out_shape=jax.ShapeDtypeStruct((rows, 8), jnp.float32),
    )(e2_p, m0_blk, b0_t.reshape(1, 128), m1_blk, b1_t.reshape(1, 128),
      m2_blk, b2_t.reshape(1, 8))


_MESH = plsc.VectorSubcoreMesh(core_axis_name="c", subcore_axis_name="s")


def _edge_sweep(wid, src_hbm, dst_hbm, ps_sh, pd_sh, ea_hbm, out_hbm,
                idx_s, idx_d, rows_s, rows_d, acc, sem, agg_sh=None):
    idx_row0 = wid * (EPWH // 128)

    def _step(t, carry):
        r0 = idx_row0 + t * KB
        pltpu.sync_copy(src_hbm.at[pl.ds(r0, KB)], idx_s)
        pltpu.sync_copy(dst_hbm.at[pl.ds(r0, KB)], idx_d)
        cps = []
        for j in range(KB):
            cps.append(pltpu.async_copy(
                ps_sh.at[idx_s.at[j]], rows_s.at[pl.ds(j * 128, 128)], sem))
            cps.append(pltpu.async_copy(
                pd_sh.at[idx_d.at[j]], rows_d.at[pl.ds(j * 128, 128)], sem))
        ebase = wid * EPWH + t * BLK
        pltpu.sync_copy(ea_hbm.at[pl.ds(ebase, BLK)], acc)
        for c in cps:
            c.wait()

        def _compute(i, carry2):
            acc[i] = jnp.maximum(acc[i] + rows_s[i] + rows_d[i], 0.0)
            return carry2
        lax.fori_loop(0, BLK, _compute, 0, unroll=4)

        pltpu.sync_copy(acc, out_hbm.at[pl.ds(ebase, BLK)])
        if agg_sh is not None:
            for j in range(KB):
                pltpu.sync_copy(acc.at[pl.ds(j * 128, 128)],
                                agg_sh.at[idx_d.at[j]], add=True)
        return carry
    lax.fori_loop(0, EPWH // BLK, _step, 0)


def _sc_phase1_body(src_hbm, dst_hbm, ps_hbm, pd_hbm, ea_hbm,
                    e1_hbm, agg2_hbm,
                    idx_s, idx_d, rows_s, rows_d, acc,
                    ps_sh, pd_sh, agg_sh, sem):
    cid = lax.axis_index("c")
    sid = lax.axis_index("s")
    wid = sid * NC + cid

    pltpu.sync_copy(ps_hbm.at[pl.ds(sid * RPT, RPT)],
                    ps_sh.at[pl.ds(sid * RPT, RPT)])
    pltpu.sync_copy(pd_hbm.at[pl.ds(sid * RPT, RPT)],
                    pd_sh.at[pl.ds(sid * RPT, RPT)])
    def _zero(i, carry):
        acc[i] = jnp.zeros((16,), jnp.float32)
        return carry
    lax.fori_loop(0, RPT, _zero, 0, unroll=4)
    pltpu.sync_copy(acc.at[pl.ds(0, RPT)], agg_sh.at[pl.ds(sid * RPT, RPT)])
    plsc.subcore_barrier()

    _edge_sweep(wid, src_hbm, dst_hbm, ps_sh, pd_sh, ea_hbm, e1_hbm,
                idx_s, idx_d, rows_s, rows_d, acc, sem, agg_sh=agg_sh)

    plsc.subcore_barrier()
    pltpu.sync_copy(agg_sh.at[pl.ds(sid * RPT, RPT)], acc.at[pl.ds(0, RPT)])
    pltpu.sync_copy(acc.at[pl.ds(0, RPT)],
                    agg2_hbm.at[cid, pl.ds(sid * RPT, RPT)])


_sc_phase1 = pl.kernel(
    _sc_phase1_body,
    out_type=(
        jax.ShapeDtypeStruct((HALF, D_EDGE), jnp.float32),
        jax.ShapeDtypeStruct((NC, NPAD, D_EDGE), jnp.float32),
    ),
    mesh=_MESH,
    scratch_types=[
        pltpu.VMEM((KB, 128), jnp.int32),
        pltpu.VMEM((KB, 128), jnp.int32),
        pltpu.VMEM((BLK, D_EDGE), jnp.float32),
        pltpu.VMEM((BLK, D_EDGE), jnp.float32),
        pltpu.VMEM((BLK, D_EDGE), jnp.float32),
        pltpu.VMEM_SHARED((NPAD, D_EDGE), jnp.float32),
        pltpu.VMEM_SHARED((NPAD, D_EDGE), jnp.float32),
        pltpu.VMEM_SHARED((NPAD, D_EDGE), jnp.float32),
        pltpu.SemaphoreType.DMA,
    ],
    compiler_params=pltpu.CompilerParams(use_tc_tiling_on_sc=False),
)


def _sc_phase2_body(src_hbm, dst_hbm, ps_hbm, pd_hbm, ee_hbm, e2_hbm,
                    idx_s, idx_d, rows_s, rows_d, acc, ps_sh, pd_sh, sem):
    cid = lax.axis_index("c")
    sid = lax.axis_index("s")
    wid = sid * NC + cid
    pltpu.sync_copy(ps_hbm.at[pl.ds(sid * RPT, RPT)],
                    ps_sh.at[pl.ds(sid * RPT, RPT)])
    pltpu.sync_copy(pd_hbm.at[pl.ds(sid * RPT, RPT)],
                    pd_sh.at[pl.ds(sid * RPT, RPT)])
    plsc.subcore_barrier()

    _edge_sweep(wid, src_hbm, dst_hbm, ps_sh, pd_sh, ee_hbm, e2_hbm,
                idx_s, idx_d, rows_s, rows_d, acc, sem)


_sc_phase2 = pl.kernel(
    _sc_phase2_body,
    out_type=jax.ShapeDtypeStruct((HALF, D_EDGE), jnp.float32),
    mesh=_MESH,
    scratch_types=[
        pltpu.VMEM((KB, 128), jnp.int32),
        pltpu.VMEM((KB, 128), jnp.int32),
        pltpu.VMEM((BLK, D_EDGE), jnp.float32),
        pltpu.VMEM((BLK, D_EDGE), jnp.float32),
        pltpu.VMEM((BLK, D_EDGE), jnp.float32),
        pltpu.VMEM_SHARED((NPAD, D_EDGE), jnp.float32),
        pltpu.VMEM_SHARED((NPAD, D_EDGE), jnp.float32),
        pltpu.SemaphoreType.DMA,
    ],
    compiler_params=pltpu.CompilerParams(use_tc_tiling_on_sc=False),
)


@jax.jit
def kernel(edge_index, x, edge_attr,
           We0, be0, Wn0, bn0, We1, be1, Wn1, bn1,
           M0, bm0, M1, bm1, M2, bm2):
    x = x.astype(jnp.float32)
    pad_e = EPAD - N_EDGES
    src = jnp.concatenate(
        [edge_index[0], jnp.full((pad_e,), N_NODES, jnp.int32)]
    ).reshape(EPAD // 128, 128)
    dst = jnp.concatenate(
        [edge_index[1], jnp.full((pad_e,), N_NODES, jnp.int32)]
    ).reshape(EPAD // 128, 128)
    hr = HALF // 128
    srch = tuple(src[k * hr:(k + 1) * hr] for k in range(NCHUNK))
    dsth = tuple(dst[k * hr:(k + 1) * hr] for k in range(NCHUNK))
    x_pad = jnp.pad(x, ((0, NPAD - N_NODES), (0, 0)))
    ea_ph = tuple(
        edge_attr[k * HALF:(k + 1) * HALF].reshape(HALF // 8, 128)
        if (k + 1) * HALF <= N_EDGES else
        jnp.pad(edge_attr[k * HALF:].reshape((N_EDGES - k * HALF) // 8, 128),
                ((0, ((k + 1) * HALF - N_EDGES) // 8), (0, 0)))
        for k in range(NCHUNK)
    )

    w0sd = jnp.concatenate([We0[:D_NODE], We0[D_NODE:2 * D_NODE]], axis=1)
    w1sd = jnp.concatenate([We1[:D_NODE], We1[D_NODE:2 * D_NODE]], axis=1)
    eye8 = jnp.eye(8, dtype=jnp.float32)
    w0e = jnp.kron(eye8, We0[2 * D_NODE:])
    b0e = jnp.tile(be0, 8)
    w1e = jnp.kron(eye8, We1[2 * D_NODE:])
    b1e = jnp.tile(be1, 8)

    km0 = jnp.kron(eye8, M0)
    km1 = jnp.kron(eye8, M1)
    km2 = jnp.kron(eye8, M2)
    bt0 = jnp.tile(bm0, 8)
    bt1 = jnp.tile(bm1, 8)
    bt2 = jnp.tile(bm2, 8)

    p0s, p0d = _node_proj(x_pad, w0sd)
    ea0_h = [_edge_proj_packed(ea_ph[k], w0e, b0e) for k in range(NCHUNK)]
    e1_h = [None] * NCHUNK
    agg2_h = [None] * NCHUNK
    for k in range(NCHUNK):
        e1_h[k], agg2_h[k] = _sc_phase1(
            srch[k], dsth[k], p0s, p0d, ea0_h[k].reshape(HALF, D_EDGE))
    p1s, p1d = _layer1_prep(agg2_h, x_pad,
                            Wn0[:D_NODE], Wn0[D_NODE:], bn0, w1sd)
    ee1_h = [_edge_proj_packed(e1_h[k].reshape(HALF // 8, 128), w1e, b1e)
             for k in range(NCHUNK)]
    out_h = []
    for k in range(NCHUNK):
        e2 = _sc_phase2(srch[k], dsth[k], p1s, p1d,
                        ee1_h[k].reshape(HALF, D_EDGE))
        out_h.append(_mlp_head(e2.reshape(HALF // 8, 128),
                               km0, bt0, km1, bt1, km2, bt2))
    out = jnp.concatenate(out_h, axis=0)
    return out.reshape(EPAD)[:N_EDGES]


# speedup vs baseline: 1.1539x; 1.0048x over previous
import functools
import jax
import jax.numpy as jnp
from jax import lax
from jax.experimental import pallas as pl
from jax.experimental.pallas import tpu as pltpu
from jax.experimental.pallas import tpu_sc as plsc

N_NODES = 10000
N_EDGES = 320000
D_NODE = 128
D_EDGE = 16

NC = 2
NS = 16
NW = NC * NS
EPAD = 327680
NCHUNK = 2
HALF = EPAD // NCHUNK
EPWH = HALF // NW
NPAD = 10112
KB = 10
BLK = KB * 128
RPT = NPAD // NS
EBLK = 8192


def _node_proj_body(x_ref, w_ref, ps_ref, pd_ref):
    p = jnp.dot(x_ref[...], w_ref[...], preferred_element_type=jnp.float32)
    ps_ref[...] = p[:, :D_EDGE]
    pd_ref[...] = p[:, D_EDGE:]


def _node_proj(x_pad, w_sd):
    return pl.pallas_call(
        _node_proj_body,
        out_shape=(
            jax.ShapeDtypeStruct((NPAD, D_EDGE), jnp.float32),
            jax.ShapeDtypeStruct((NPAD, D_EDGE), jnp.float32),
        ),
    )(x_pad, w_sd)


def _edge_proj_packed_body(a_ref, w_ref, b_ref, o_ref):
    o_ref[...] = (
        jnp.dot(a_ref[...], w_ref[...], preferred_element_type=jnp.float32)
        + b_ref[...]
    )


def _edge_proj_packed(a_p, w_blk, b_tiled):
    rows = a_p.shape[0]
    return pl.pallas_call(
        _edge_proj_packed_body,
        grid=(rows // (EBLK // 8),),
        in_specs=[
            pl.BlockSpec((EBLK // 8, 128), lambda i: (i, 0)),
            pl.BlockSpec((128, 128), lambda i: (0, 0)),
            pl.BlockSpec((1, 128), lambda i: (0, 0)),
        ],
        out_specs=pl.BlockSpec((EBLK // 8, 128), lambda i: (i, 0)),
        out_shape=jax.ShapeDtypeStruct((rows, 128), jnp.float32),
    )(a_p, w_blk, b_tiled.reshape(1, 128))


def _layer1_prep_body(*refs):
    agg_refs = refs[:NCHUNK]
    x_ref, wnx_ref, wna_ref, bn_ref, wsd_ref, ps_ref, pd_ref = refs[NCHUNK:]
    agg = sum(r[0] + r[1] for r in agg_refs)
    x1 = jnp.maximum(
        jnp.dot(x_ref[...], wnx_ref[...], preferred_element_type=jnp.float32)
        + jnp.dot(agg, wna_ref[...], preferred_element_type=jnp.float32)
        + bn_ref[...],
        0.0,
    )
    p = jnp.dot(x1, wsd_ref[...], preferred_element_type=jnp.float32)
    ps_ref[...] = p[:, :D_EDGE]
    pd_ref[...] = p[:, D_EDGE:]


def _layer1_prep(agg2s, x_pad, wnx, wna, bn, w_sd):
    return pl.pallas_call(
        _layer1_prep_body,
        out_shape=(
            jax.ShapeDtypeStruct((NPAD, D_EDGE), jnp.float32),
            jax.ShapeDtypeStruct((NPAD, D_EDGE), jnp.float32),
        ),
    )(*agg2s, x_pad, wnx, wna, bn.reshape(1, D_NODE), w_sd)


def _mlp_body(e_ref, m0_ref, b0_ref, m1_ref, b1_ref, m2_ref, b2_ref, o_ref):
    h = jnp.maximum(
        jnp.dot(e_ref[...], m0_ref[...], preferred_element_type=jnp.float32)
        + b0_ref[...], 0.0)
    h = jnp.maximum(
        jnp.dot(h, m1_ref[...], preferred_element_type=jnp.float32)
        + b1_ref[...], 0.0)
    o_ref[...] = (
        jnp.dot(h, m2_ref[...], preferred_element_type=jnp.float32)
        + b2_ref[...]
    )


def _mlp_head(e2_p, m0_blk, b0_t, m1_blk, b1_t, m2_blk, b2_t):
    rows = e2_p.shape[0]
    return pl.pallas_call(
        _mlp_body,
        grid=(rows // (EBLK // 8),),
        in_specs=[
            pl.BlockSpec((EBLK // 8, 128), lambda i: (i, 0)),
            pl.BlockSpec((128, 128), lambda i: (0, 0)),
            pl.BlockSpec((1, 128), lambda i: (0, 0)),
            pl.BlockSpec((128, 128), lambda i: (0, 0)),
            pl.BlockSpec((1, 128), lambda i: (0, 0)),
            pl.BlockSpec((128, 8), lambda i: (0, 0)),
            pl.BlockSpec((1, 8), lambda i: (0, 0)),
        ],
        out_specs=pl.BlockSpec((EBLK // 8, 8), lambda i: (i, 0)),
        out_shape=jax.ShapeDtypeStruct((rows, 8), jnp.float32),
    )(e2_p, m0_blk, b0_t.reshape(1, 128), m1_blk, b1_t.reshape(1, 128),
      m2_blk, b2_t.reshape(1, 8))


_MESH = plsc.VectorSubcoreMesh(core_axis_name="c", subcore_axis_name="s")


def _edge_sweep(wid, src_hbm, dst_hbm, ps_sh, pd_sh, ea_hbm, out_hbm,
                idx_s, idx_d, rows_s, rows_d, acc, sem, agg_sh=None):
    idx_row0 = wid * (EPWH // 128)

    def _step(t, carry):
        r0 = idx_row0 + t * KB
        pltpu.sync_copy(src_hbm.at[pl.ds(r0, KB)], idx_s)
        pltpu.sync_copy(dst_hbm.at[pl.ds(r0, KB)], idx_d)
        cps = []
        for j in range(KB):
            cps.append(pltpu.async_copy(
                ps_sh.at[idx_s.at[j]], rows_s.at[pl.ds(j * 128, 128)], sem))
            cps.append(pltpu.async_copy(
                pd_sh.at[idx_d.at[j]], rows_d.at[pl.ds(j * 128, 128)], sem))
        ebase = wid * EPWH + t * BLK
        pltpu.sync_copy(ea_hbm.at[pl.ds(ebase, BLK)], acc)
        for c in cps:
            c.wait()

        def _compute(i, carry2):
            acc[i] = jnp.maximum(acc[i] + rows_s[i] + rows_d[i], 0.0)
            return carry2
        lax.fori_loop(0, BLK, _compute, 0, unroll=4)

        pltpu.sync_copy(acc, out_hbm.at[pl.ds(ebase, BLK)])
        if agg_sh is not None:
            sts = [pltpu.async_copy(acc.at[pl.ds(j * 128, 128)],
                                    agg_sh.at[idx_d.at[j]], sem, add=True)
                   for j in range(KB)]
            for d in sts:
                d.wait()
        return carry
    lax.fori_loop(0, EPWH // BLK, _step, 0)


def _sc_phase1_body(src_hbm, dst_hbm, ps_hbm, pd_hbm, ea_hbm,
                    e1_hbm, agg2_hbm,
                    idx_s, idx_d, rows_s, rows_d, acc,
                    ps_sh, pd_sh, agg_sh, sem):
    cid = lax.axis_index("c")
    sid = lax.axis_index("s")
    wid = sid * NC + cid

    pltpu.sync_copy(ps_hbm.at[pl.ds(sid * RPT, RPT)],
                    ps_sh.at[pl.ds(sid * RPT, RPT)])
    pltpu.sync_copy(pd_hbm.at[pl.ds(sid * RPT, RPT)],
                    pd_sh.at[pl.ds(sid * RPT, RPT)])
    def _zero(i, carry):
        acc[i] = jnp.zeros((16,), jnp.float32)
        return carry
    lax.fori_loop(0, RPT, _zero, 0, unroll=4)
    pltpu.sync_copy(acc.at[pl.ds(0, RPT)], agg_sh.at[pl.ds(sid * RPT, RPT)])
    plsc.subcore_barrier()

    _edge_sweep(wid, src_hbm, dst_hbm, ps_sh, pd_sh, ea_hbm, e1_hbm,
                idx_s, idx_d, rows_s, rows_d, acc, sem, agg_sh=agg_sh)

    plsc.subcore_barrier()
    pltpu.sync_copy(agg_sh.at[pl.ds(sid * RPT, RPT)], acc.at[pl.ds(0, RPT)])
    pltpu.sync_copy(acc.at[pl.ds(0, RPT)],
                    agg2_hbm.at[cid, pl.ds(sid * RPT, RPT)])


_sc_phase1 = pl.kernel(
    _sc_phase1_body,
    out_type=(
        jax.ShapeDtypeStruct((HALF, D_EDGE), jnp.float32),
        jax.ShapeDtypeStruct((NC, NPAD, D_EDGE), jnp.float32),
    ),
    mesh=_MESH,
    scratch_types=[
        pltpu.VMEM((KB, 128), jnp.int32),
        pltpu.VMEM((KB, 128), jnp.int32),
        pltpu.VMEM((BLK, D_EDGE), jnp.float32),
        pltpu.VMEM((BLK, D_EDGE), jnp.float32),
        pltpu.VMEM((BLK, D_EDGE), jnp.float32),
        pltpu.VMEM_SHARED((NPAD, D_EDGE), jnp.float32),
        pltpu.VMEM_SHARED((NPAD, D_EDGE), jnp.float32),
        pltpu.VMEM_SHARED((NPAD, D_EDGE), jnp.float32),
        pltpu.SemaphoreType.DMA,
    ],
    compiler_params=pltpu.CompilerParams(use_tc_tiling_on_sc=False),
)


def _sc_phase2_body(src_hbm, dst_hbm, ps_hbm, pd_hbm, ee_hbm, e2_hbm,
                    idx_s, idx_d, rows_s, rows_d, acc, ps_sh, pd_sh, sem):
    cid = lax.axis_index("c")
    sid = lax.axis_index("s")
    wid = sid * NC + cid
    pltpu.sync_copy(ps_hbm.at[pl.ds(sid * RPT, RPT)],
                    ps_sh.at[pl.ds(sid * RPT, RPT)])
    pltpu.sync_copy(pd_hbm.at[pl.ds(sid * RPT, RPT)],
                    pd_sh.at[pl.ds(sid * RPT, RPT)])
    plsc.subcore_barrier()

    _edge_sweep(wid, src_hbm, dst_hbm, ps_sh, pd_sh, ee_hbm, e2_hbm,
                idx_s, idx_d, rows_s, rows_d, acc, sem)


_sc_phase2 = pl.kernel(
    _sc_phase2_body,
    out_type=jax.ShapeDtypeStruct((HALF, D_EDGE), jnp.float32),
    mesh=_MESH,
    scratch_types=[
        pltpu.VMEM((KB, 128), jnp.int32),
        pltpu.VMEM((KB, 128), jnp.int32),
        pltpu.VMEM((BLK, D_EDGE), jnp.float32),
        pltpu.VMEM((BLK, D_EDGE), jnp.float32),
        pltpu.VMEM((BLK, D_EDGE), jnp.float32),
        pltpu.VMEM_SHARED((NPAD, D_EDGE), jnp.float32),
        pltpu.VMEM_SHARED((NPAD, D_EDGE), jnp.float32),
        pltpu.SemaphoreType.DMA,
    ],
    compiler_params=pltpu.CompilerParams(use_tc_tiling_on_sc=False),
)


@jax.jit
def kernel(edge_index, x, edge_attr,
           We0, be0, Wn0, bn0, We1, be1, Wn1, bn1,
           M0, bm0, M1, bm1, M2, bm2):
    x = x.astype(jnp.float32)
    pad_e = EPAD - N_EDGES
    src = jnp.concatenate(
        [edge_index[0], jnp.full((pad_e,), N_NODES, jnp.int32)]
    ).reshape(EPAD // 128, 128)
    dst = jnp.concatenate(
        [edge_index[1], jnp.full((pad_e,), N_NODES, jnp.int32)]
    ).reshape(EPAD // 128, 128)
    hr = HALF // 128
    srch = tuple(src[k * hr:(k + 1) * hr] for k in range(NCHUNK))
    dsth = tuple(dst[k * hr:(k + 1) * hr] for k in range(NCHUNK))
    x_pad = jnp.pad(x, ((0, NPAD - N_NODES), (0, 0)))
    ea_ph = tuple(
        edge_attr[k * HALF:(k + 1) * HALF].reshape(HALF // 8, 128)
        if (k + 1) * HALF <= N_EDGES else
        jnp.pad(edge_attr[k * HALF:].reshape((N_EDGES - k * HALF) // 8, 128),
                ((0, ((k + 1) * HALF - N_EDGES) // 8), (0, 0)))
        for k in range(NCHUNK)
    )

    w0sd = jnp.concatenate([We0[:D_NODE], We0[D_NODE:2 * D_NODE]], axis=1)
    w1sd = jnp.concatenate([We1[:D_NODE], We1[D_NODE:2 * D_NODE]], axis=1)
    eye8 = jnp.eye(8, dtype=jnp.float32)
    w0e = jnp.kron(eye8, We0[2 * D_NODE:])
    b0e = jnp.tile(be0, 8)
    w1e = jnp.kron(eye8, We1[2 * D_NODE:])
    b1e = jnp.tile(be1, 8)

    km0 = jnp.kron(eye8, M0)
    km1 = jnp.kron(eye8, M1)
    km2 = jnp.kron(eye8, M2)
    bt0 = jnp.tile(bm0, 8)
    bt1 = jnp.tile(bm1, 8)
    bt2 = jnp.tile(bm2, 8)

    p0s, p0d = _node_proj(x_pad, w0sd)
    ea0_h = [_edge_proj_packed(ea_ph[k], w0e, b0e) for k in range(NCHUNK)]
    e1_h = [None] * NCHUNK
    agg2_h = [None] * NCHUNK
    for k in range(NCHUNK):
        e1_h[k], agg2_h[k] = _sc_phase1(
            srch[k], dsth[k], p0s, p0d, ea0_h[k].reshape(HALF, D_EDGE))
    p1s, p1d = _layer1_prep(agg2_h, x_pad,
                            Wn0[:D_NODE], Wn0[D_NODE:], bn0, w1sd)
    ee1_h = [_edge_proj_packed(e1_h[k].reshape(HALF // 8, 128), w1e, b1e)
             for k in range(NCHUNK)]
    out_h = []
    for k in range(NCHUNK):
        e2 = _sc_phase2(srch[k], dsth[k], p1s, p1d,
                        ee1_h[k].reshape(HALF, D_EDGE))
        out_h.append(_mlp_head(e2.reshape(HALF // 8, 128),
                               km0, bt0, km1, bt1, km2, bt2))
    out = jnp.concatenate(out_h, axis=0)
    return out.reshape(EPAD)[:N_EDGES]


# speedup vs baseline: 1.1540x; 1.0001x over previous
import jax
import jax.numpy as jnp
from jax import lax
from jax.experimental import pallas as pl
from jax.experimental.pallas import tpu as pltpu
from jax.experimental.pallas import tpu_sc as plsc

N_NODES = 10000
N_EDGES = 320000
D_NODE = 128
D_EDGE = 16

NC = 2
NS = 16
NW = NC * NS
EPAD = 327680
NCHUNK = 2
HALF = EPAD // NCHUNK
EPWH = HALF // NW
NPAD = 10112
KB = 10
BLK = KB * 128
RPT = NPAD // NS
EBLK = 8192


def _node_proj_body(x_ref, w_ref, ps_ref, pd_ref):
    p = jnp.dot(x_ref[...], w_ref[...], preferred_element_type=jnp.float32)
    ps_ref[...] = p[:, :D_EDGE]
    pd_ref[...] = p[:, D_EDGE:]


def _node_proj(x_pad, w_sd):
    return pl.pallas_call(
        _node_proj_body,
        out_shape=(
            jax.ShapeDtypeStruct((NPAD, D_EDGE), jnp.float32),
            jax.ShapeDtypeStruct((NPAD, D_EDGE), jnp.float32),
        ),
    )(x_pad, w_sd)


def _edge_proj_packed_body(a_ref, w_ref, b_ref, o_ref):
    o_ref[...] = (
        jnp.dot(a_ref[...], w_ref[...], preferred_element_type=jnp.float32)
        + b_ref[...]
    )


def _edge_proj_packed(a_p, w_blk, b_tiled):
    rows = a_p.shape[0]
    return pl.pallas_call(
        _edge_proj_packed_body,
        grid=(rows // (EBLK // 8),),
        in_specs=[
            pl.BlockSpec((EBLK // 8, 128), lambda i: (i, 0)),
            pl.BlockSpec((128, 128), lambda i: (0, 0)),
            pl.BlockSpec((1, 128), lambda i: (0, 0)),
        ],
        out_specs=pl.BlockSpec((EBLK // 8, 128), lambda i: (i, 0)),
        out_shape=jax.ShapeDtypeStruct((rows, 128), jnp.float32),
    )(a_p, w_blk, b_tiled.reshape(1, 128))


def _layer1_prep_body(*refs):
    agg_refs = refs[:NCHUNK]
    x_ref, wnx_ref, wna_ref, bn_ref, wsd_ref, ps_ref, pd_ref = refs[NCHUNK:]
    agg = sum(r[0] + r[1] for r in agg_refs)
    x1 = jnp.maximum(
        jnp.dot(x_ref[...], wnx_ref[...], preferred_element_type=jnp.float32)
        + jnp.dot(agg, wna_ref[...], preferred_element_type=jnp.float32)
        + bn_ref[...],
        0.0,
    )
    p = jnp.dot(x1, wsd_ref[...], preferred_element_type=jnp.float32)
    ps_ref[...] = p[:, :D_EDGE]
    pd_ref[...] = p[:, D_EDGE:]


def _layer1_prep(agg2s, x_pad, wnx, wna, bn, w_sd):
    return pl.pallas_call(
        _layer1_prep_body,
        out_shape=(
            jax.ShapeDtypeStruct((NPAD, D_EDGE), jnp.float32),
            jax.ShapeDtypeStruct((NPAD, D_EDGE), jnp.float32),
        ),
    )(*agg2s, x_pad, wnx, wna, bn.reshape(1, D_NODE), w_sd)


def _mlp_body(e_ref, m0_ref, b0_ref, m1_ref, b1_ref, m2_ref, b2_ref, o_ref):
    h = jnp.maximum(
        jnp.dot(e_ref[...], m0_ref[...], preferred_element_type=jnp.float32)
        + b0_ref[...], 0.0)
    h = jnp.maximum(
        jnp.dot(h, m1_ref[...], preferred_element_type=jnp.float32)
        + b1_ref[...], 0.0)
    o_ref[...] = (
        jnp.dot(h, m2_ref[...], preferred_element_type=jnp.float32)
        + b2_ref[...]
    )


def _mlp_head(e2_p, m0_blk, b0_t, m1_blk, b1_t, m2_blk, b2_t):
    rows = e2_p.shape[0]
    return pl.pallas_call(
        _mlp_body,
        grid=(rows // (EBLK // 8),),
        in_specs=[
            pl.BlockSpec((EBLK // 8, 128), lambda i: (i, 0)),
            pl.BlockSpec((128, 128), lambda i: (0, 0)),
            pl.BlockSpec((1, 128), lambda i: (0, 0)),
            pl.BlockSpec((128, 128), lambda i: (0, 0)),
            pl.BlockSpec((1, 128), lambda i: (0, 0)),
            pl.BlockSpec((128, 8), lambda i: (0, 0)),
            pl.BlockSpec((1, 8), lambda i: (0, 0)),
        ],
        out_specs=pl.BlockSpec((EBLK // 8, 8), lambda i: (i, 0)),
        out_shape=jax.ShapeDtypeStruct((rows, 8), jnp.float32),
    )(e2_p, m0_blk, b0_t.reshape(1, 128), m1_blk, b1_t.reshape(1, 128),
      m2_blk, b2_t.reshape(1, 8))


_MESH = plsc.VectorSubcoreMesh(core_axis_name="c", subcore_axis_name="s")


def _edge_sweep(wid, src_hbm, dst_hbm, ps_sh, pd_sh, ea_hbm, out_hbm,
                idx_s, idx_d, rows_s, rows_d, acc, sem, agg_sh=None):
    idx_row0 = wid * (EPWH // 128)

    def _step(t, carry):
        r0 = idx_row0 + t * KB
        pltpu.sync_copy(src_hbm.at[pl.ds(r0, KB)], idx_s)
        pltpu.sync_copy(dst_hbm.at[pl.ds(r0, KB)], idx_d)
        cps = []
        for j in range(KB):
            cps.append(pltpu.async_copy(
                ps_sh.at[idx_s.at[j]], rows_s.at[pl.ds(j * 128, 128)], sem))
            cps.append(pltpu.async_copy(
                pd_sh.at[idx_d.at[j]], rows_d.at[pl.ds(j * 128, 128)], sem))
        ebase = wid * EPWH + t * BLK
        pltpu.sync_copy(ea_hbm.at[pl.ds(ebase, BLK)], acc)
        for c in cps:
            c.wait()

        def _compute(i, carry2):
            acc[i] = jnp.maximum(acc[i] + rows_s[i] + rows_d[i], 0.0)
            return carry2
        lax.fori_loop(0, BLK, _compute, 0, unroll=4)

        pltpu.sync_copy(acc, out_hbm.at[pl.ds(ebase, BLK)])
        if agg_sh is not None:
            sts = [pltpu.async_copy(acc.at[pl.ds(j * 128, 128)],
                                    agg_sh.at[idx_d.at[j]], sem, add=True)
                   for j in range(KB)]
            for d in sts:
                d.wait()
        return carry
    lax.fori_loop(0, EPWH // BLK, _step, 0)


def _sc_phase1_body(src_hbm, dst_hbm, ps_hbm, pd_hbm, ea_hbm,
                    e1_hbm, agg2_hbm,
                    idx_s, idx_d, rows_s, rows_d, acc,
                    ps_sh, pd_sh, agg_sh, sem):
    cid = lax.axis_index("c")
    sid = lax.axis_index("s")
    wid = sid * NC + cid

    pltpu.sync_copy(ps_hbm.at[pl.ds(sid * RPT, RPT)],
                    ps_sh.at[pl.ds(sid * RPT, RPT)])
    pltpu.sync_copy(pd_hbm.at[pl.ds(sid * RPT, RPT)],
                    pd_sh.at[pl.ds(sid * RPT, RPT)])
    def _zero(i, carry):
        acc[i] = jnp.zeros((16,), jnp.float32)
        return carry
    lax.fori_loop(0, RPT, _zero, 0, unroll=4)
    pltpu.sync_copy(acc.at[pl.ds(0, RPT)], agg_sh.at[pl.ds(sid * RPT, RPT)])
    plsc.subcore_barrier()

    _edge_sweep(wid, src_hbm, dst_hbm, ps_sh, pd_sh, ea_hbm, e1_hbm,
                idx_s, idx_d, rows_s, rows_d, acc, sem, agg_sh=agg_sh)

    plsc.subcore_barrier()
    pltpu.sync_copy(agg_sh.at[pl.ds(sid * RPT, RPT)], acc.at[pl.ds(0, RPT)])
    pltpu.sync_copy(acc.at[pl.ds(0, RPT)],
                    agg2_hbm.at[cid, pl.ds(sid * RPT, RPT)])


_sc_phase1 = pl.kernel(
    _sc_phase1_body,
    out_type=(
        jax.ShapeDtypeStruct((HALF, D_EDGE), jnp.float32),
        jax.ShapeDtypeStruct((NC, NPAD, D_EDGE), jnp.float32),
    ),
    mesh=_MESH,
    scratch_types=[
        pltpu.VMEM((KB, 128), jnp.int32),
        pltpu.VMEM((KB, 128), jnp.int32),
        pltpu.VMEM((BLK, D_EDGE), jnp.float32),
        pltpu.VMEM((BLK, D_EDGE), jnp.float32),
        pltpu.VMEM((BLK, D_EDGE), jnp.float32),
        pltpu.VMEM_SHARED((NPAD, D_EDGE), jnp.float32),
        pltpu.VMEM_SHARED((NPAD, D_EDGE), jnp.float32),
        pltpu.VMEM_SHARED((NPAD, D_EDGE), jnp.float32),
        pltpu.SemaphoreType.DMA,
    ],
    compiler_params=pltpu.CompilerParams(use_tc_tiling_on_sc=False),
)


def _sc_phase2_body(src_hbm, dst_hbm, ps_hbm, pd_hbm, ee_hbm, e2_hbm,
                    idx_s, idx_d, rows_s, rows_d, acc, ps_sh, pd_sh, sem):
    cid = lax.axis_index("c")
    sid = lax.axis_index("s")
    wid = sid * NC + cid
    pltpu.sync_copy(ps_hbm.at[pl.ds(sid * RPT, RPT)],
                    ps_sh.at[pl.ds(sid * RPT, RPT)])
    pltpu.sync_copy(pd_hbm.at[pl.ds(sid * RPT, RPT)],
                    pd_sh.at[pl.ds(sid * RPT, RPT)])
    plsc.subcore_barrier()

    _edge_sweep(wid, src_hbm, dst_hbm, ps_sh, pd_sh, ee_hbm, e2_hbm,
                idx_s, idx_d, rows_s, rows_d, acc, sem)


_sc_phase2 = pl.kernel(
    _sc_phase2_body,
    out_type=jax.ShapeDtypeStruct((HALF, D_EDGE), jnp.float32),
    mesh=_MESH,
    scratch_types=[
        pltpu.VMEM((KB, 128), jnp.int32),
        pltpu.VMEM((KB, 128), jnp.int32),
        pltpu.VMEM((BLK, D_EDGE), jnp.float32),
        pltpu.VMEM((BLK, D_EDGE), jnp.float32),
        pltpu.VMEM((BLK, D_EDGE), jnp.float32),
        pltpu.VMEM_SHARED((NPAD, D_EDGE), jnp.float32),
        pltpu.VMEM_SHARED((NPAD, D_EDGE), jnp.float32),
        pltpu.SemaphoreType.DMA,
    ],
    compiler_params=pltpu.CompilerParams(use_tc_tiling_on_sc=False),
)


@jax.jit
def kernel(edge_index, x, edge_attr,
           We0, be0, Wn0, bn0, We1, be1, Wn1, bn1,
           M0, bm0, M1, bm1, M2, bm2):
    x = x.astype(jnp.float32)
    pad_e = EPAD - N_EDGES
    src = jnp.concatenate(
        [edge_index[0], jnp.full((pad_e,), N_NODES, jnp.int32)]
    ).reshape(EPAD // 128, 128)
    dst = jnp.concatenate(
        [edge_index[1], jnp.full((pad_e,), N_NODES, jnp.int32)]
    ).reshape(EPAD // 128, 128)
    hr = HALF // 128
    srch = tuple(src[k * hr:(k + 1) * hr] for k in range(NCHUNK))
    dsth = tuple(dst[k * hr:(k + 1) * hr] for k in range(NCHUNK))
    x_pad = jnp.pad(x, ((0, NPAD - N_NODES), (0, 0)))
    ea_ph = tuple(
        edge_attr[k * HALF:(k + 1) * HALF].reshape(HALF // 8, 128)
        if (k + 1) * HALF <= N_EDGES else
        jnp.pad(edge_attr[k * HALF:].reshape((N_EDGES - k * HALF) // 8, 128),
                ((0, ((k + 1) * HALF - N_EDGES) // 8), (0, 0)))
        for k in range(NCHUNK)
    )

    w0sd = jnp.concatenate([We0[:D_NODE], We0[D_NODE:2 * D_NODE]], axis=1)
    w1sd = jnp.concatenate([We1[:D_NODE], We1[D_NODE:2 * D_NODE]], axis=1)
    eye8 = jnp.eye(8, dtype=jnp.float32)
    w0e = jnp.kron(eye8, We0[2 * D_NODE:])
    b0e = jnp.tile(be0, 8)
    w1e = jnp.kron(eye8, We1[2 * D_NODE:])
    b1e = jnp.tile(be1, 8)

    km0 = jnp.kron(eye8, M0)
    km1 = jnp.kron(eye8, M1)
    km2 = jnp.kron(eye8, M2)
    bt0 = jnp.tile(bm0, 8)
    bt1 = jnp.tile(bm1, 8)
    bt2 = jnp.tile(bm2, 8)

    p0s, p0d = _node_proj(x_pad, w0sd)
    ea0_h = [_edge_proj_packed(ea_ph[k], w0e, b0e) for k in range(NCHUNK)]
    e1_h = [None] * NCHUNK
    agg2_h = [None] * NCHUNK
    for k in range(NCHUNK):
        e1_h[k], agg2_h[k] = _sc_phase1(
            srch[k], dsth[k], p0s, p0d, ea0_h[k].reshape(HALF, D_EDGE))
    p1s, p1d = _layer1_prep(agg2_h, x_pad,
                            Wn0[:D_NODE], Wn0[D_NODE:], bn0, w1sd)
    ee1_h = [_edge_proj_packed(e1_h[k].reshape(HALF // 8, 128), w1e, b1e)
             for k in range(NCHUNK)]
    out_h = []
    for k in range(NCHUNK):
        e2 = _sc_phase2(srch[k], dsth[k], p1s, p1d,
                        ee1_h[k].reshape(HALF, D_EDGE))
        out_h.append(_mlp_head(e2.reshape(HALF // 8, 128),
                               km0, bt0, km1, bt1, km2, bt2))
    out = jnp.concatenate(out_h, axis=0)
    return out.reshape(EPAD)[:N_EDGES]
